# baseline (speedup 1.0000x reference)
import jax
import jax.numpy as jnp
from jax import lax
from jax.experimental import pallas as pl
from jax.experimental.pallas import tpu as pltpu
from jax.experimental.pallas import tpu_sc as plsc

D = 64
NC = 2
NS = 16
NW = NC * NS
B = 4096
S = 200
LANES = 128
VROWS = 500000


def _emb_body(xT_hbm, tab_hbm, out_hbm, idx_v, idx2_v, buf, tb, sem_g, sem_w):
    w = lax.axis_index("s") * NC + lax.axis_index("c")
    base = w * LANES
    pltpu.sync_copy(xT_hbm.at[:, pl.ds(base, LANES)], idx_v)

    def prep(g, carry):
        row = g // 8
        col = (g % 8) * 16
        v = idx_v[row, pl.ds(col, 16)]
        idx2_v[row, pl.ds(col, 16)] = lax.shift_right_logical(v, 1)
        return carry

    lax.fori_loop(0, S * 8, prep, 0)

    def fire(s, q):
        pltpu.async_copy(tab_hbm.at[idx2_v.at[s]], buf.at[q], sem_g)

    fire(0, 0)
    rbase = lax.iota(jnp.int32, 16)

    def step(s, carry):
        p = lax.rem(s, 2)

        @pl.when(s + 1 < S)
        def _():
            fire(s + 1, 1 - p)

        pltpu.make_async_copy(tab_hbm.at[pl.ds(0, LANES)], buf.at[p], sem_g).wait()

        @pl.when(s >= 2)
        def _():
            pltpu.make_async_copy(
                tb.at[0], out_hbm.at[0, :, pl.ds(0, LANES)], sem_w
            ).wait()

        pv16 = jnp.full((16,), p, jnp.int32)
        for rg in range(8):
            pc = (idx_v[s, pl.ds(rg * 16, 16)] & 1) * D
            rv = rbase + rg * 16
            for d in range(D):
                gval = plsc.load_gather(buf, [pv16, rv, pc + d])
                tb[p, d, pl.ds(rg * 16, 16)] = gval

        pltpu.async_copy(tb.at[p], out_hbm.at[s, :, pl.ds(base, LANES)], sem_w)
        return carry

    lax.fori_loop(0, S, step, 0)
    pltpu.make_async_copy(tb.at[0], out_hbm.at[0, :, pl.ds(0, LANES)], sem_w).wait()
    pltpu.make_async_copy(tb.at[0], out_hbm.at[0, :, pl.ds(0, LANES)], sem_w).wait()


def kernel(x, table):
    xT = jnp.transpose(x.astype(jnp.int32), (1, 0))
    tab = table.reshape(VROWS, 128)
    mesh = plsc.VectorSubcoreMesh(core_axis_name="c", subcore_axis_name="s")
    out = pl.kernel(
        _emb_body,
        out_type=jax.ShapeDtypeStruct((S, D, B), jnp.float32),
        mesh=mesh,
        scratch_types=[
            pltpu.VMEM((S, LANES), jnp.int32),
            pltpu.VMEM((S, LANES), jnp.int32),
            pltpu.VMEM((2, LANES, 128), jnp.float32),
            pltpu.VMEM((2, D, LANES), jnp.float32),
            pltpu.SemaphoreType.DMA,
            pltpu.SemaphoreType.DMA,
        ],
        compiler_params=pltpu.CompilerParams(
            use_tc_tiling_on_sc=True, needs_layout_passes=False
        ),
    )(xT, tab)
    return jnp.transpose(out, (2, 0, 1))


# speedup vs baseline: 1.3725x; 1.3725x over previous
import jax
import jax.numpy as jnp
from jax import lax
from jax.experimental import pallas as pl
from jax.experimental.pallas import tpu as pltpu
from jax.experimental.pallas import tpu_sc as plsc

D = 64
NC = 2
NS = 16
NW = NC * NS
B = 4096
S = 200
LANES = 128
VROWS = 500000


def _emb_body(xT_hbm, tab_hbm, out_hbm, idx_v, idx2_v, buf, tb, sem_g, sem_w):
    w = lax.axis_index("s") * NC + lax.axis_index("c")
    base = w * LANES
    pltpu.sync_copy(xT_hbm.at[:, pl.ds(base, LANES)], idx_v)

    def prep(g, carry):
        row = g // 8
        col = (g % 8) * 16
        v = idx_v[row, pl.ds(col, 16)]
        idx2_v[row, pl.ds(col, 16)] = lax.shift_right_logical(v, 1)
        return carry

    lax.fori_loop(0, S * 8, prep, 0)

    def fire(s, q):
        pltpu.async_copy(tab_hbm.at[idx2_v.at[s]], buf.at[q], sem_g)

    fire(0, 0)
    rbase = lax.iota(jnp.int32, 16)

    def transpose_block(s, bufp, tbp):
        for rg in range(8):
            pc = (idx_v[s, pl.ds(rg * 16, 16)] & 1) * D
            rv = rbase + rg * 16
            for dg in range(8):
                gs = [
                    plsc.load_gather(bufp, [rv, pc + (dg * 8 + j)])
                    for j in range(8)
                ]
                for j in range(8):
                    tbp[dg * 8 + j, pl.ds(rg * 16, 16)] = gs[j]
        pltpu.async_copy(tbp, out_hbm.at[s, :, pl.ds(base, LANES)], sem_w)

    def step(s, carry):
        p = lax.rem(s, 2)

        @pl.when(s + 1 < S)
        def _():
            fire(s + 1, 1 - p)

        pltpu.make_async_copy(tab_hbm.at[pl.ds(0, LANES)], buf.at[p], sem_g).wait()

        @pl.when(s >= 2)
        def _():
            pltpu.make_async_copy(
                tb.at[0], out_hbm.at[0, :, pl.ds(0, LANES)], sem_w
            ).wait()

        @pl.when(p == 0)
        def _():
            transpose_block(s, buf.at[0], tb.at[0])

        @pl.when(p == 1)
        def _():
            transpose_block(s, buf.at[1], tb.at[1])

        return carry

    lax.fori_loop(0, S, step, 0)
    pltpu.make_async_copy(tb.at[0], out_hbm.at[0, :, pl.ds(0, LANES)], sem_w).wait()
    pltpu.make_async_copy(tb.at[0], out_hbm.at[0, :, pl.ds(0, LANES)], sem_w).wait()


def kernel(x, table):
    xT = jnp.transpose(x.astype(jnp.int32), (1, 0))
    tab = table.reshape(VROWS, 128)
    mesh = plsc.VectorSubcoreMesh(core_axis_name="c", subcore_axis_name="s")
    out = pl.kernel(
        _emb_body,
        out_type=jax.ShapeDtypeStruct((S, D, B), jnp.float32),
        mesh=mesh,
        scratch_types=[
            pltpu.VMEM((S, LANES), jnp.int32),
            pltpu.VMEM((S, LANES), jnp.int32),
            pltpu.VMEM((2, LANES, 128), jnp.float32),
            pltpu.VMEM((2, D, LANES), jnp.float32),
            pltpu.SemaphoreType.DMA,
            pltpu.SemaphoreType.DMA,
        ],
        compiler_params=pltpu.CompilerParams(
            use_tc_tiling_on_sc=True, needs_layout_passes=False
        ),
    )(xT, tab)
    return jnp.transpose(out, (2, 0, 1))
